# baseline (speedup 1.0000x reference)
import jax
import jax.numpy as jnp
from jax import lax
from jax.experimental import pallas as pl
from jax.experimental.pallas import tpu as pltpu
from jax.experimental.pallas import tpu_sc as plsc

DIM = 64
PAD_DIM = 128
WINDOW = 128
NC = 2
NS = 16
NW = NC * NS


def _gather_kernel(num_indices):
    chunks = num_indices // WINDOW
    cpw = chunks // NW
    mesh = plsc.VectorSubcoreMesh(core_axis_name="c", subcore_axis_name="s")

    @pl.kernel(
        out_type=jax.ShapeDtypeStruct((num_indices, PAD_DIM), jnp.float32),
        mesh=mesh,
        scratch_types=[
            pltpu.VMEM((cpw, WINDOW), jnp.int32),
            pltpu.VMEM((WINDOW, PAD_DIM), jnp.float32),
            pltpu.SemaphoreType.DMA,
        ],
    )
    def kern(table_hbm, idx_hbm, out_hbm, idx_v, rows_v, sem):
        wid = lax.axis_index("s") * NC + lax.axis_index("c")
        pltpu.sync_copy(idx_hbm.at[wid], idx_v)

        @pl.loop(0, cpw)
        def _(j):
            pltpu.async_copy(table_hbm.at[idx_v.at[j]], rows_v, sem).wait()
            base = (wid * cpw + j) * WINDOW
            pltpu.sync_copy(rows_v, out_hbm.at[pl.ds(base, WINDOW)])

    return kern


def _depad_kernel(pad_ref, out_ref):
    br, s, _ = out_ref.shape
    out_ref[...] = pad_ref[:, :DIM].reshape(br, s, DIM)


def _depad(res, b, s):
    br = 128
    return pl.pallas_call(
        _depad_kernel,
        grid=(b // br,),
        in_specs=[pl.BlockSpec((br * s, PAD_DIM), lambda i: (i, 0))],
        out_specs=pl.BlockSpec((br, s, DIM), lambda i: (i, 0, 0)),
        out_shape=jax.ShapeDtypeStruct((b, s, DIM), jnp.float32),
    )(res)


def kernel(x, weight):
    b, s = x.shape
    n = b * s
    idx = x.reshape(NW, n // (NW * WINDOW), WINDOW).astype(jnp.int32)
    table = jnp.pad(weight, ((0, 0), (0, PAD_DIM - DIM)))
    out = _gather_kernel(n)(table, idx)
    return _depad(out, b, s)


# speedup vs baseline: 1.1027x; 1.1027x over previous
import jax
import jax.numpy as jnp
from jax import lax
from jax.experimental import pallas as pl
from jax.experimental.pallas import tpu as pltpu
from jax.experimental.pallas import tpu_sc as plsc

DIM = 64
PAIR_DIM = 128
WINDOW = 128
NC = 2
NS = 16
NW = NC * NS


def _gather_kernel(num_indices):
    chunks = num_indices // WINDOW
    cpw = chunks // NW
    mesh = plsc.VectorSubcoreMesh(core_axis_name="c", subcore_axis_name="s")

    @pl.kernel(
        out_type=jax.ShapeDtypeStruct((num_indices, PAIR_DIM), jnp.float32),
        mesh=mesh,
        scratch_types=[
            pltpu.VMEM((cpw, WINDOW), jnp.int32),
            pltpu.VMEM((WINDOW, PAIR_DIM), jnp.float32),
            pltpu.SemaphoreType.DMA,
        ],
    )
    def kern(table_hbm, idx_hbm, out_hbm, idx_v, rows_v, sem):
        wid = lax.axis_index("s") * NC + lax.axis_index("c")
        pltpu.sync_copy(idx_hbm.at[wid], idx_v)

        @pl.loop(0, cpw)
        def _(j):
            pltpu.async_copy(table_hbm.at[idx_v.at[j]], rows_v, sem).wait()
            base = (wid * cpw + j) * WINDOW
            pltpu.sync_copy(rows_v, out_hbm.at[pl.ds(base, WINDOW)])

    return kern


def _tail_kernel(res_ref, xt_ref, o_ref):
    data = res_ref[...]
    par = (xt_ref[0, 0] & 1)[:, None] == 1
    sel = jnp.where(par, data[:, DIM:], data[:, :DIM])
    o_ref[0] = sel.T


def _tail(res, xt, b, s):
    return pl.pallas_call(
        _tail_kernel,
        grid=(s,),
        in_specs=[
            pl.BlockSpec((b, PAIR_DIM), lambda i: (i, 0)),
            pl.BlockSpec((1, 1, b), lambda i: (i, 0, 0)),
        ],
        out_specs=pl.BlockSpec((1, DIM, b), lambda i: (i, 0, 0)),
        out_shape=jax.ShapeDtypeStruct((s, DIM, b), jnp.float32),
    )(res, xt.reshape(s, 1, b))


def kernel(x, weight):
    b, s = x.shape
    n = b * s
    xt = x.T.astype(jnp.int32)
    idx = (xt >> 1).reshape(NW, n // (NW * WINDOW), WINDOW)
    table = weight.reshape(weight.shape[0] // 2, PAIR_DIM)
    res = _gather_kernel(n)(table, idx)
    out_t = _tail(res, xt, b, s)
    return jnp.transpose(out_t, (2, 0, 1))
